# Initial kernel scaffold; baseline (speedup 1.0000x reference)
import jax
import jax.numpy as jnp
from jax.experimental import pallas as pl


def kernel(x, edge_index, batch, params):
    raise NotImplementedError("write your pallas kernel here")


# speedup vs baseline: 5.6456x; 5.6456x over previous
import functools

import jax
import jax.numpy as jnp
from jax import lax
from jax.experimental import pallas as pl
from jax.experimental.pallas import tpu as pltpu
from jax.experimental.pallas import tpu_sc as plsc

_N = 10000
_E = 320000
_H = 128
_B = 64
_HOPS = 5

_NC = 2
_NS = 16
_EDGES_PER_TILE = _E // (_NC * _NS)
_CH = 128
_FULL_CHUNKS = _EDGES_PER_TILE // _CH
_TAIL = _EDGES_PER_TILE - _FULL_CHUNKS * _CH
_N_PAD = 10240
_ROWS_PER_TILE = _N_PAD // _NS
_ZROWS = 128


def _gelu(x):
    return 0.5 * x * (1.0 + lax.erf(x * 0.7071067811865476))


def _ffnn(x, w1, b1, w2, b2):
    return _gelu(jnp.dot(x, w1) + b1) @ w2 + b2


def _sc_agg_body(h_hbm, src_hbm, dst_hbm, out_hbm,
                 agg_sp, src_v, dst_v, rows_v, src_t, dst_t, rows_t, zrow, sem):
    c = lax.axis_index("c")
    s = lax.axis_index("s")

    def _zfill(i, carry):
        for j in range(_H // 16):
            zrow[i, pl.ds(j * 16, 16)] = jnp.zeros((16,), jnp.float32)
        return carry
    lax.fori_loop(0, _ZROWS, _zfill, 0)
    for k in range(_ROWS_PER_TILE // _ZROWS):
        pltpu.sync_copy(zrow, agg_sp.at[pl.ds(s * _ROWS_PER_TILE + k * _ZROWS, _ZROWS)])
    plsc.subcore_barrier()

    ebase = c * (_E // _NC) + s * _EDGES_PER_TILE

    def _chunk(k, carry):
        off = ebase + k * _CH
        pltpu.sync_copy(src_hbm.at[pl.ds(off, _CH)], src_v)
        pltpu.sync_copy(dst_hbm.at[pl.ds(off, _CH)], dst_v)
        pltpu.async_copy(h_hbm.at[src_v], rows_v, sem).wait()
        pltpu.sync_copy(rows_v, agg_sp.at[dst_v], add=True)
        return carry
    lax.fori_loop(0, _FULL_CHUNKS, _chunk, 0)

    toff = ebase + _FULL_CHUNKS * _CH
    pltpu.sync_copy(src_hbm.at[pl.ds(toff, _TAIL)], src_t)
    pltpu.sync_copy(dst_hbm.at[pl.ds(toff, _TAIL)], dst_t)
    pltpu.async_copy(h_hbm.at[src_t], rows_t, sem).wait()
    pltpu.sync_copy(rows_t, agg_sp.at[dst_t], add=True)

    plsc.subcore_barrier()
    pltpu.sync_copy(agg_sp.at[pl.ds(s * _ROWS_PER_TILE, _ROWS_PER_TILE)],
                    out_hbm.at[c].at[pl.ds(s * _ROWS_PER_TILE, _ROWS_PER_TILE)])


@functools.cache
def _sc_agg_kernel():
    return pl.kernel(
        _sc_agg_body,
        out_type=jax.ShapeDtypeStruct((_NC, _N_PAD, _H), jnp.float32),
        mesh=plsc.VectorSubcoreMesh(core_axis_name="c", subcore_axis_name="s",
                                    num_cores=_NC),
        scratch_types=[
            pltpu.VMEM_SHARED((_N_PAD, _H), jnp.float32),
            pltpu.VMEM((_CH,), jnp.int32),
            pltpu.VMEM((_CH,), jnp.int32),
            pltpu.VMEM((_CH, _H), jnp.float32),
            pltpu.VMEM((_TAIL,), jnp.int32),
            pltpu.VMEM((_TAIL,), jnp.int32),
            pltpu.VMEM((_TAIL, _H), jnp.float32),
            pltpu.VMEM((_ZROWS, _H), jnp.float32),
            pltpu.SemaphoreType.DMA,
        ],
    )


def _sc_agg(h, src, dst):
    return _sc_agg_kernel()(h, src, dst)


def _pre_body(x_ref, w1_ref, b1_ref, w2_ref, b2_ref, o_ref):
    o_ref[...] = _ffnn(x_ref[...], w1_ref[...], b1_ref[...],
                       w2_ref[...], b2_ref[...])


def _pre_call(x, p):
    return pl.pallas_call(
        _pre_body,
        out_shape=jax.ShapeDtypeStruct((_N, _H), jnp.float32),
    )(x, p["W1"], p["b1"].reshape(1, -1), p["W2"], p["b2"].reshape(1, -1))


def _hop_body(h_ref, parts_ref, brow_ref, bcol_ref,
              vn_ref, uw1_ref, ub1_ref, uw2_ref, ub2_ref,
              pw1_ref, pb1_ref, pw2_ref, pb2_ref,
              gw1_ref, gb1_ref, gw2_ref, gb2_ref,
              fw1_ref, fb1_ref, fw2_ref, fb2_ref,
              bng_ref, bnb_ref, o_ref):
    h = h_ref[...]

    oh_t = (lax.broadcasted_iota(jnp.int32, (_B, _N), 0)
            == brow_ref[...]).astype(jnp.float32)
    cnt = jnp.sum(oh_t, axis=1, keepdims=True)
    pool = jnp.dot(oh_t, h) / jnp.maximum(cnt, 1.0)

    vn = vn_ref[...] + _ffnn(pool, uw1_ref[...], ub1_ref[...],
                             uw2_ref[...], ub2_ref[...])
    outvn = _ffnn(vn, pw1_ref[...], pb1_ref[...],
                  pw2_ref[...], pb2_ref[...])

    agg = parts_ref[0, :_N, :] + parts_ref[1, :_N, :]
    h1 = _ffnn(h + agg, gw1_ref[...], gb1_ref[...],
               gw2_ref[...], gb2_ref[...]) + h

    m = jnp.mean(h1, axis=0, keepdims=True)
    v = jnp.mean((h1 - m) ** 2, axis=0, keepdims=True)
    h1 = (h1 - m) / jnp.sqrt(v + 1e-5) * bng_ref[...] + bnb_ref[...]

    oh_n = (lax.broadcasted_iota(jnp.int32, (_N, _B), 1)
            == bcol_ref[...]).astype(jnp.float32)
    gath = jnp.dot(oh_n, outvn)

    o_ref[...] = _ffnn(gath + h1, fw1_ref[...], fb1_ref[...],
                       fw2_ref[...], fb2_ref[...]) + h1


def _hop_call(h, parts, brow, bcol, vn, upd, prop, gin, ffnn, bng, bnb):
    return pl.pallas_call(
        _hop_body,
        out_shape=jax.ShapeDtypeStruct((_N, _H), jnp.float32),
    )(h, parts, brow, bcol, vn,
      upd["W1"], upd["b1"].reshape(1, -1), upd["W2"], upd["b2"].reshape(1, -1),
      prop["W1"], prop["b1"].reshape(1, -1), prop["W2"], prop["b2"].reshape(1, -1),
      gin["W1"], gin["b1"].reshape(1, -1), gin["W2"], gin["b2"].reshape(1, -1),
      ffnn["W1"], ffnn["b1"].reshape(1, -1), ffnn["W2"], ffnn["b2"].reshape(1, -1),
      bng.reshape(1, -1), bnb.reshape(1, -1))


def _post_body(h_ref, brow_ref, w1_ref, b1_ref, w2_ref, b2_ref, o_ref):
    h = h_ref[...]
    oh_t = (lax.broadcasted_iota(jnp.int32, (_B, _N), 0)
            == brow_ref[...]).astype(jnp.float32)
    cnt = jnp.sum(oh_t, axis=1, keepdims=True)
    pool = jnp.dot(oh_t, h) / jnp.maximum(cnt, 1.0)
    o_ref[...] = _ffnn(pool, w1_ref[...], b1_ref[...], w2_ref[...], b2_ref[...])


def _post_call(h, brow, p):
    return pl.pallas_call(
        _post_body,
        out_shape=jax.ShapeDtypeStruct((_B, _H), jnp.float32),
    )(h, brow, p["W1"], p["b1"].reshape(1, -1), p["W2"], p["b2"].reshape(1, -1))


def kernel(x, edge_index, batch, params):
    src = edge_index[0].astype(jnp.int32)
    dst = edge_index[1].astype(jnp.int32)
    brow = batch.reshape(1, _N).astype(jnp.int32)
    bcol = batch.reshape(_N, 1).astype(jnp.int32)

    h = _pre_call(x, params["pre"])
    for i in range(_HOPS):
        parts = _sc_agg(h, src, dst)
        h = _hop_call(h, parts, brow, bcol, params["vn"],
                      params["upd"][i], params["prop"][i],
                      params["gin"][i], params["ffnn"][i],
                      params["bn_g"][i], params["bn_b"][i])
    return _post_call(h, brow, params["post"])
